# baseline (speedup 1.0000x reference)
import functools

import jax
import jax.numpy as jnp
from jax import lax
from jax.experimental import pallas as pl
from jax.experimental.pallas import tpu as pltpu
from jax.experimental.pallas import tpu_sc as plsc

N = 10000
D = 128
E = 320000

NC = 2
NS = 16
NW = NC * NS
CHUNK = 128
NCH = 80
EPW = CHUNK * NCH
EPAD = NW * EPW
SLAB = 640
LAST_SLAB = N - SLAB * (NS - 1)


def _sc_body(x_hbm, src_hbm, dst_hbm, vals_hbm, z_hbm, out_hbm,
             acc, srcv, dstv, valsv, rowsb, sem):
    c = lax.axis_index("c")
    s = lax.axis_index("s")
    wid = c * NS + s
    cbase = wid * NCH
    rbase = s * SLAB

    @pl.when(s < NS - 1)
    def _():
        pltpu.sync_copy(z_hbm, acc.at[pl.ds(rbase, SLAB)])

    @pl.when(s == NS - 1)
    def _():
        pltpu.sync_copy(z_hbm.at[pl.ds(0, LAST_SLAB)],
                        acc.at[pl.ds(rbase, LAST_SLAB)])

    pltpu.sync_copy(src_hbm.at[pl.ds(cbase, NCH)], srcv)
    pltpu.sync_copy(dst_hbm.at[pl.ds(cbase, NCH)], dstv)
    pltpu.sync_copy(vals_hbm.at[pl.ds(cbase, NCH)], valsv)

    plsc.subcore_barrier()

    def chunk_body(i, carry):
        pltpu.async_copy(x_hbm.at[srcv.at[i]], rowsb, sem).wait()

        for g in range(CHUNK // 16):
            vv = valsv[i, pl.ds(g * 16, 16)]
            for t in range(16):
                e = g * 16 + t
                vb = jnp.full((16,), vv[t], dtype=jnp.float32)
                for q in range(D // 16):
                    sl = pl.ds(q * 16, 16)
                    rowsb[e, sl] = rowsb[e, sl] * vb

        pltpu.sync_copy(rowsb, acc.at[dstv.at[i]], add=True)
        return carry

    lax.fori_loop(0, NCH, chunk_body, 0)

    plsc.subcore_barrier()

    @pl.when(s < NS - 1)
    def _():
        pltpu.sync_copy(acc.at[pl.ds(rbase, SLAB)],
                        out_hbm.at[c, pl.ds(rbase, SLAB)])

    @pl.when(s == NS - 1)
    def _():
        pltpu.sync_copy(acc.at[pl.ds(rbase, LAST_SLAB)],
                        out_hbm.at[c, pl.ds(rbase, LAST_SLAB)])


def _sc_segment_sum(x, src3, dst3, vals3, zrows):
    mesh = plsc.VectorSubcoreMesh(core_axis_name="c", subcore_axis_name="s")
    fn = functools.partial(
        pl.kernel,
        out_type=jax.ShapeDtypeStruct((NC, N, D), jnp.float32),
        mesh=mesh,
        scratch_types=[
            pltpu.VMEM_SHARED((N, D), jnp.float32),
            pltpu.VMEM((NCH, CHUNK), jnp.int32),
            pltpu.VMEM((NCH, CHUNK), jnp.int32),
            pltpu.VMEM((NCH, CHUNK), jnp.float32),
            pltpu.VMEM((CHUNK, D), jnp.float32),
            pltpu.SemaphoreType.DMA,
        ],
    )(_sc_body)
    return fn(x, src3, dst3, vals3, zrows)


BLK = 1000


def _tc_finalize(partial, W, b2):
    def body(p_ref, w_ref, b_ref, o_ref):
        s = p_ref[0] + p_ref[1]
        y = jnp.dot(s, w_ref[...], preferred_element_type=jnp.float32)
        o_ref[...] = jnp.maximum(y + b_ref[...], 0.0)

    return pl.pallas_call(
        body,
        grid=(N // BLK,),
        in_specs=[
            pl.BlockSpec((2, BLK, D), lambda i: (0, i, 0)),
            pl.BlockSpec((D, D), lambda i: (0, 0)),
            pl.BlockSpec((1, D), lambda i: (0, 0)),
        ],
        out_specs=pl.BlockSpec((BLK, D), lambda i: (i, 0)),
        out_shape=jax.ShapeDtypeStruct((N, D), jnp.float32),
    )(partial, W, b2)


def kernel(x, edge_index, edge_vals, W, b):
    pad = EPAD - E
    src = jnp.pad(edge_index[0].astype(jnp.int32), (0, pad))
    dst = jnp.pad(edge_index[1].astype(jnp.int32), (0, pad))
    vals_p = jnp.pad(edge_vals, (0, pad))
    src3 = src.reshape(-1, CHUNK)
    dst3 = dst.reshape(-1, CHUNK)
    vals3 = vals_p.reshape(-1, CHUNK)
    zrows = jnp.zeros((SLAB, D), jnp.float32)
    partial = _sc_segment_sum(x, src3, dst3, vals3, zrows)
    return _tc_finalize(partial, W, b.reshape(1, D))


# speedup vs baseline: 1.2009x; 1.2009x over previous
import functools

import jax
import jax.numpy as jnp
from jax import lax
from jax.experimental import pallas as pl
from jax.experimental.pallas import tpu as pltpu
from jax.experimental.pallas import tpu_sc as plsc

N = 10000
D = 128
E = 320000

NC = 2
NS = 16
NW = NC * NS
CH = 64
NROW = 80
NP = NROW
EPW = 2 * CH * NROW
EPAD = NW * EPW
SLAB = 640
LAST_SLAB = N - SLAB * (NS - 1)


def _sc_body(x_hbm, src_hbm, dst_hbm, vals_hbm, z_hbm, out_hbm,
             acc, srcv, dstv, valsv, rows0, rows1, srcb0, srcb1,
             dstb0, dstb1, sem0, sem1):
    c = lax.axis_index("c")
    s = lax.axis_index("s")
    wid = c * NS + s
    cbase = wid * NROW
    rbase = s * SLAB

    @pl.when(s < NS - 1)
    def _():
        pltpu.sync_copy(z_hbm, acc.at[pl.ds(rbase, SLAB)])

    @pl.when(s == NS - 1)
    def _():
        pltpu.sync_copy(z_hbm.at[pl.ds(0, LAST_SLAB)],
                        acc.at[pl.ds(rbase, LAST_SLAB)])

    pltpu.sync_copy(src_hbm.at[pl.ds(cbase, NROW)], srcv)
    pltpu.sync_copy(dst_hbm.at[pl.ds(cbase, NROW)], dstv)
    pltpu.sync_copy(vals_hbm.at[pl.ds(cbase, NROW)], valsv)

    plsc.subcore_barrier()

    def copy_idx(row, half, sb, db):
        for k in range(CH // 16):
            sb[pl.ds(k * 16, 16)] = srcv[row, pl.ds(half * CH + k * 16, 16)]
            db[pl.ds(k * 16, 16)] = dstv[row, pl.ds(half * CH + k * 16, 16)]

    def gather_start(sb, rb, sem):
        pltpu.async_copy(x_hbm.at[sb], rb, sem)

    def gather_wait(sb, rb, sem):
        pltpu.make_async_copy(x_hbm.at[sb], rb, sem).wait()

    def scale(row, half, rb):
        for g in range(CH // 16):
            vv = valsv[row, pl.ds(half * CH + g * 16, 16)]
            for t in range(16):
                e = g * 16 + t
                vb = jnp.full((16,), vv[t], dtype=jnp.float32)
                for q in range(D // 16):
                    sl = pl.ds(q * 16, 16)
                    rb[e, sl] = rb[e, sl] * vb

    def scatter(rb, db):
        pltpu.sync_copy(rb, acc.at[db], add=True)

    copy_idx(0, 0, srcb0, dstb0)
    gather_start(srcb0, rows0, sem0)
    copy_idx(0, 1, srcb1, dstb1)

    def body(p, carry):
        gather_start(srcb1, rows1, sem1)
        gather_wait(srcb0, rows0, sem0)
        scale(p, 0, rows0)
        scatter(rows0, dstb0)

        @pl.when(p < NP - 1)
        def _():
            copy_idx(p + 1, 0, srcb0, dstb0)
            gather_start(srcb0, rows0, sem0)

        gather_wait(srcb1, rows1, sem1)
        scale(p, 1, rows1)
        scatter(rows1, dstb1)

        @pl.when(p < NP - 1)
        def _():
            copy_idx(p + 1, 1, srcb1, dstb1)

        return carry

    lax.fori_loop(0, NP, body, 0)

    plsc.subcore_barrier()

    @pl.when(s < NS - 1)
    def _():
        pltpu.sync_copy(acc.at[pl.ds(rbase, SLAB)],
                        out_hbm.at[c, pl.ds(rbase, SLAB)])

    @pl.when(s == NS - 1)
    def _():
        pltpu.sync_copy(acc.at[pl.ds(rbase, LAST_SLAB)],
                        out_hbm.at[c, pl.ds(rbase, LAST_SLAB)])


def _sc_segment_sum(x, src3, dst3, vals3, zrows):
    mesh = plsc.VectorSubcoreMesh(core_axis_name="c", subcore_axis_name="s")
    fn = functools.partial(
        pl.kernel,
        out_type=jax.ShapeDtypeStruct((NC, N, D), jnp.float32),
        mesh=mesh,
        scratch_types=[
            pltpu.VMEM_SHARED((N, D), jnp.float32),
            pltpu.VMEM((NROW, 2 * CH), jnp.int32),
            pltpu.VMEM((NROW, 2 * CH), jnp.int32),
            pltpu.VMEM((NROW, 2 * CH), jnp.float32),
            pltpu.VMEM((CH, D), jnp.float32),
            pltpu.VMEM((CH, D), jnp.float32),
            pltpu.VMEM((CH,), jnp.int32),
            pltpu.VMEM((CH,), jnp.int32),
            pltpu.VMEM((CH,), jnp.int32),
            pltpu.VMEM((CH,), jnp.int32),
            pltpu.SemaphoreType.DMA,
            pltpu.SemaphoreType.DMA,
        ],
    )(_sc_body)
    return fn(x, src3, dst3, vals3, zrows)


BLK = 1000


def _tc_finalize(partial, W, b2):
    def body(p_ref, w_ref, b_ref, o_ref):
        s = p_ref[0] + p_ref[1]
        y = jnp.dot(s, w_ref[...], preferred_element_type=jnp.float32)
        o_ref[...] = jnp.maximum(y + b_ref[...], 0.0)

    return pl.pallas_call(
        body,
        grid=(N // BLK,),
        in_specs=[
            pl.BlockSpec((2, BLK, D), lambda i: (0, i, 0)),
            pl.BlockSpec((D, D), lambda i: (0, 0)),
            pl.BlockSpec((1, D), lambda i: (0, 0)),
        ],
        out_specs=pl.BlockSpec((BLK, D), lambda i: (i, 0)),
        out_shape=jax.ShapeDtypeStruct((N, D), jnp.float32),
    )(partial, W, b2)


def kernel(x, edge_index, edge_vals, W, b):
    pad = EPAD - E
    src = jnp.pad(edge_index[0].astype(jnp.int32), (0, pad))
    dst = jnp.pad(edge_index[1].astype(jnp.int32), (0, pad))
    vals_p = jnp.pad(edge_vals, (0, pad))
    src3 = src.reshape(-1, 2 * CH)
    dst3 = dst.reshape(-1, 2 * CH)
    vals3 = vals_p.reshape(-1, 2 * CH)
    zrows = jnp.zeros((SLAB, D), jnp.float32)
    partial = _sc_segment_sum(x, src3, dst3, vals3, zrows)
    return _tc_finalize(partial, W, b.reshape(1, D))
